# Initial kernel scaffold; baseline (speedup 1.0000x reference)
import jax
import jax.numpy as jnp
from jax.experimental import pallas as pl


def kernel(x, pos, edge_index, W1, b1, W2, b2, G1, g1, G2, g2, G3, g3):
    raise NotImplementedError("write your pallas kernel here")


# speedup vs baseline: 1.6675x; 1.6675x over previous
import functools

import jax
import jax.numpy as jnp
from jax.experimental import pallas as pl

_INTERPRET = False

N_NODES = 10000
D_X = 128
D_H = 64


def _node_pre_body(x_ref, pos_ref, w1xt_ref, w1pt_ref, b1_ref, w2t_ref,
                   u_ref, w_ref, self_ref):
    xb = x_ref[...]
    pb = pos_ref[...]
    w_blk = jnp.dot(pb, w1pt_ref[...], preferred_element_type=jnp.float32)
    ux = jnp.dot(xb, w1xt_ref[...], preferred_element_type=jnp.float32)
    u_blk = ux + w_blk + b1_ref[...]
    u_ref[...] = u_blk
    w_ref[...] = w_blk
    self_ref[...] = jnp.dot(jax.nn.relu(ux + b1_ref[...]), w2t_ref[...],
                            preferred_element_type=jnp.float32)


def _node_pre(x, pos, w1xt, w1pt, b1, w2t, bn=1000):
    n = x.shape[0]
    grid = (n // bn,)
    return pl.pallas_call(
        _node_pre_body,
        grid=grid,
        in_specs=[
            pl.BlockSpec((bn, D_X), lambda i: (i, 0)),
            pl.BlockSpec((bn, 3), lambda i: (i, 0)),
            pl.BlockSpec((D_X, D_H), lambda i: (0, 0)),
            pl.BlockSpec((3, D_H), lambda i: (0, 0)),
            pl.BlockSpec((1, D_H), lambda i: (0, 0)),
            pl.BlockSpec((D_H, D_H), lambda i: (0, 0)),
        ],
        out_specs=[
            pl.BlockSpec((bn, D_H), lambda i: (i, 0)),
            pl.BlockSpec((bn, D_H), lambda i: (i, 0)),
            pl.BlockSpec((bn, D_H), lambda i: (i, 0)),
        ],
        out_shape=[
            jax.ShapeDtypeStruct((n, D_H), jnp.float32),
            jax.ShapeDtypeStruct((n, D_H), jnp.float32),
            jax.ShapeDtypeStruct((n, D_H), jnp.float32),
        ],
        interpret=_INTERPRET,
    )(x, pos, w1xt, w1pt, b1.reshape(1, D_H), w2t)


def _edge_mlp_body(t_ref, w2t_ref, h_ref):
    h_ref[...] = jnp.dot(jax.nn.relu(t_ref[...]), w2t_ref[...],
                         preferred_element_type=jnp.float32)


def _edge_mlp(t, w2t, be=2000):
    e = t.shape[0]
    grid = (e // be,)
    return pl.pallas_call(
        _edge_mlp_body,
        grid=grid,
        in_specs=[
            pl.BlockSpec((be, D_H), lambda i: (i, 0)),
            pl.BlockSpec((D_H, D_H), lambda i: (0, 0)),
        ],
        out_specs=pl.BlockSpec((be, D_H), lambda i: (i, 0)),
        out_shape=jax.ShapeDtypeStruct((e, D_H), jnp.float32),
        interpret=_INTERPRET,
    )(t, w2t)


def _global_mlp_body(a_ref, b2_ref, g1t_ref, g1_ref, g2t_ref, g2_ref,
                     g3t_ref, g3_ref, o_ref):
    a = a_ref[...] + b2_ref[...]
    a = jax.nn.relu(jnp.dot(a, g1t_ref[...], preferred_element_type=jnp.float32)
                    + g1_ref[...])
    a = jax.nn.relu(jnp.dot(a, g2t_ref[...], preferred_element_type=jnp.float32)
                    + g2_ref[...])
    o_ref[...] = jnp.dot(a, g3t_ref[...], preferred_element_type=jnp.float32) \
        + g3_ref[...]


def _global_mlp(agg, b2, g1t, g1, g2t, g2, g3t, g3, bn=1000):
    n = agg.shape[0]
    grid = (n // bn,)
    return pl.pallas_call(
        _global_mlp_body,
        grid=grid,
        in_specs=[
            pl.BlockSpec((bn, D_H), lambda i: (i, 0)),
            pl.BlockSpec((1, D_H), lambda i: (0, 0)),
            pl.BlockSpec((D_H, 32), lambda i: (0, 0)),
            pl.BlockSpec((1, 32), lambda i: (0, 0)),
            pl.BlockSpec((32, 128), lambda i: (0, 0)),
            pl.BlockSpec((1, 128), lambda i: (0, 0)),
            pl.BlockSpec((128, 128), lambda i: (0, 0)),
            pl.BlockSpec((1, 128), lambda i: (0, 0)),
        ],
        out_specs=pl.BlockSpec((bn, 128), lambda i: (i, 0)),
        out_shape=jax.ShapeDtypeStruct((n, 128), jnp.float32),
        interpret=_INTERPRET,
    )(agg, b2.reshape(1, D_H), g1t, g1.reshape(1, 32), g2t, g2.reshape(1, 128),
      g3t, g3.reshape(1, 128))


def kernel(x, pos, edge_index, W1, b1, W2, b2, G1, g1, G2, g2, G3, g3):
    src = edge_index[0].astype(jnp.int32)
    dst = edge_index[1].astype(jnp.int32)
    w1xt = W1[:, :D_X].T
    w1pt = W1[:, D_X:].T
    u, w, selfinit = _node_pre(x, pos, w1xt, w1pt, b1, W2.T)
    t = u[src] - w[dst]
    h = _edge_mlp(t, W2.T)
    agg = jax.ops.segment_max(h, dst, num_segments=N_NODES)
    agg = jnp.maximum(agg, selfinit)
    return _global_mlp(agg, b2, G1.T, g1, G2.T, g2, G3.T, g3)
